# Initial kernel scaffold; baseline (speedup 1.0000x reference)
import jax
import jax.numpy as jnp
from jax.experimental import pallas as pl


def kernel(x, dst_idx, dst_size):
    raise NotImplementedError("write your pallas kernel here")


# speedup vs baseline: 4.1167x; 4.1167x over previous
import jax
import jax.numpy as jnp
from jax import lax
from jax.experimental import pallas as pl
from jax.experimental.pallas import tpu as pltpu
from jax.experimental.pallas import tpu_sc as plsc

N_EDGES = 320000
D_FEAT = 128
DST_SIZE = 10000

NC = 2
NS = 16
L = 16

CH = 128
N_CHUNKS = N_EDGES // CH
ROWS_PER_CORE = DST_SIZE // NC
ROWS_PER_TILE = (ROWS_PER_CORE // NS) // 8 * 8
DUMMY_ROW = ROWS_PER_CORE
ACC_ROWS = ROWS_PER_CORE + 8
LAST_TILE_ROWS = ROWS_PER_CORE - (NS - 1) * ROWS_PER_TILE


def _extract(vec, i):
    return jnp.sum(jnp.where(lax.iota(jnp.int32, L) == i, vec, 0))


def _sc_body(x_hbm, idx_hbm, bounds_hbm, zeros_hbm, out_hbm,
             bounds_v, idx_v, idx2_v, rows_v, acc):
    c = lax.axis_index("c")
    s = lax.axis_index("s")
    r0 = c * ROWS_PER_CORE

    @pl.when(s < NS - 1)
    def _():
        pltpu.sync_copy(zeros_hbm.at[pl.ds(0, ROWS_PER_TILE)],
                        acc.at[pl.ds(s * ROWS_PER_TILE, ROWS_PER_TILE)])

    @pl.when(s == NS - 1)
    def _():
        pltpu.sync_copy(zeros_hbm.at[pl.ds(0, LAST_TILE_ROWS)],
                        acc.at[pl.ds(s * ROWS_PER_TILE, LAST_TILE_ROWS)])

    pltpu.sync_copy(bounds_hbm, bounds_v)
    bvec = bounds_v[...]
    cs = jnp.where(c == 0, 0, bvec[2])
    ce = jnp.where(c == 0, bvec[1], bvec[3])

    plsc.subcore_barrier()

    def chunk_body(k, carry):
        j = cs + s + k * NS
        base = j * CH
        pltpu.sync_copy(x_hbm.at[pl.ds(base, CH)], rows_v)
        pltpu.sync_copy(idx_hbm.at[pl.ds(base, CH)], idx_v)
        for q in range(CH // L):
            v = jnp.minimum(idx_v[pl.ds(q * L, L)], DST_SIZE - 1)
            ok = (v >= r0) & (v < r0 + ROWS_PER_CORE)
            idx2_v[pl.ds(q * L, L)] = jnp.where(ok, v - r0, DUMMY_ROW)
        pltpu.sync_copy(rows_v, acc.at[idx2_v], add=True)
        return carry

    n_iter = jnp.maximum(ce - (cs + s) + NS - 1, 0) // NS
    lax.fori_loop(0, n_iter, chunk_body, 0)

    plsc.subcore_barrier()

    gbase = r0 + s * ROWS_PER_TILE

    @pl.when(s < NS - 1)
    def _():
        pltpu.sync_copy(acc.at[pl.ds(s * ROWS_PER_TILE, ROWS_PER_TILE)],
                        out_hbm.at[pl.ds(gbase, ROWS_PER_TILE)])

    @pl.when(s == NS - 1)
    def _():
        pltpu.sync_copy(acc.at[pl.ds(s * ROWS_PER_TILE, LAST_TILE_ROWS)],
                        out_hbm.at[pl.ds(gbase, LAST_TILE_ROWS)])


_segment_sum_sc = pl.kernel(
    _sc_body,
    out_type=jax.ShapeDtypeStruct((DST_SIZE, D_FEAT), jnp.float32),
    mesh=plsc.VectorSubcoreMesh(core_axis_name="c", subcore_axis_name="s"),
    scratch_types=[
        pltpu.VMEM((L,), jnp.int32),
        pltpu.VMEM((CH,), jnp.int32),
        pltpu.VMEM((CH,), jnp.int32),
        pltpu.VMEM((CH, D_FEAT), jnp.float32),
        pltpu.VMEM_SHARED((ACC_ROWS, D_FEAT), jnp.float32),
    ],
)


@jax.jit
def kernel(x, dst_idx, dst_size):
    del dst_size
    idx = dst_idx.astype(jnp.int32)
    split = jnp.searchsorted(idx, ROWS_PER_CORE).astype(jnp.int32)
    ce0 = (split + CH - 1) // CH
    cs1 = split // CH
    bounds = jnp.zeros((L,), jnp.int32)
    bounds = bounds.at[1].set(ce0)
    bounds = bounds.at[2].set(cs1)
    bounds = bounds.at[3].set(N_CHUNKS)
    zeros = jnp.zeros((LAST_TILE_ROWS, D_FEAT), jnp.float32)
    return _segment_sum_sc(x, idx, bounds, zeros)
